# baseline (speedup 1.0000x reference)
import functools

import jax
import jax.numpy as jnp
from jax import lax
from jax.experimental import pallas as pl
from jax.experimental.pallas import tpu as pltpu
from jax.experimental.pallas import tpu_sc as plsc

_L = 16


def _bprmf_kernel(B, D, n_workers):
    bpw = B // n_workers
    mesh = plsc.VectorSubcoreMesh(core_axis_name="c", subcore_axis_name="s")

    @functools.partial(
        pl.kernel,
        mesh=mesh,
        compiler_params=pltpu.CompilerParams(
            needs_layout_passes=False, use_tc_tiling_on_sc=False
        ),
        out_type=jax.ShapeDtypeStruct((B,), jnp.float32),
        scratch_types=[
            pltpu.VMEM((bpw,), jnp.int32),
            pltpu.VMEM((bpw,), jnp.int32),
            pltpu.VMEM((bpw, D), jnp.float32),
            pltpu.VMEM((bpw, D), jnp.float32),
            pltpu.VMEM((bpw,), jnp.float32),
            pltpu.VMEM((bpw,), jnp.float32),
            pltpu.SemaphoreType.DMA,
        ],
    )
    def run(u_hbm, i_hbm, p_hbm, q_hbm, b_hbm, out_hbm,
            uv, iv, pv, qv, bv, ov, sem):
        wid = lax.axis_index("s") * 2 + lax.axis_index("c")
        base = wid * bpw
        pltpu.sync_copy(u_hbm.at[pl.ds(base, bpw)], uv)
        pltpu.sync_copy(i_hbm.at[pl.ds(base, bpw)], iv)
        cp_p = pltpu.async_copy(p_hbm.at[uv], pv, sem)
        cp_q = pltpu.async_copy(q_hbm.at[iv], qv, sem)
        cp_b = pltpu.async_copy(b_hbm.at[iv], bv, sem)
        cp_p.wait()
        cp_q.wait()
        cp_b.wait()

        nj = D // _L
        lanes = lax.iota(jnp.int32, _L)

        def group(g, _):
            r0 = g * _L
            sum_vec = jnp.zeros((_L,), jnp.float32)
            for t in range(_L):
                r = r0 + t
                acc = pv[r, pl.ds(0, _L)] * qv[r, pl.ds(0, _L)]
                for j in range(1, nj):
                    acc = acc + pv[r, pl.ds(j * _L, _L)] * qv[r, pl.ds(j * _L, _L)]
                sum_vec = jnp.where(lanes == t, jnp.sum(acc), sum_vec)
            ov[pl.ds(r0, _L)] = sum_vec + bv[pl.ds(r0, _L)]
            return 0

        lax.fori_loop(0, bpw // _L, group, 0)
        pltpu.sync_copy(ov, out_hbm.at[pl.ds(base, bpw)])

    return run


def kernel(u, i, P, Q, bi):
    B = u.shape[0]
    D = P.shape[1]
    return _bprmf_kernel(B, D, 32)(u, i, P, Q, bi.reshape(-1))
